# baseline (speedup 1.0000x reference)
import jax
import jax.numpy as jnp
from jax.experimental import pallas as pl
from jax.experimental.pallas import tpu as pltpu

_N_ITERS = 21
_INV_TEMPERATURE = 25.0
_TINY = 1e-30
_BLOCK_B = 8


def _sinkhorn_block(x_ref, o_ref):
    for mm in range(_BLOCK_B):
        la = x_ref[mm] * _INV_TEMPERATURE
        m = jnp.max(la, axis=1, keepdims=True)
        o_ref[mm] = jnp.exp(la - m)

    def half_steps(c):
        rs, prods = [], []
        for mm in range(_BLOCK_B):
            k = o_ref[mm]
            u = jnp.sum(k * c[mm], axis=1, keepdims=True)
            r = 1.0 / jnp.maximum(u, _TINY)
            rs.append(r)
            prods.append(k * r)
        return rs, prods

    def body(_, c):
        _, prods = half_steps(c)
        news = []
        for mm in range(_BLOCK_B):
            v = jnp.sum(prods[mm], axis=0)
            news.append(1.0 / jnp.maximum(v, _TINY))
        return jnp.stack(news)

    c0 = jnp.ones((_BLOCK_B, 128), jnp.float32)
    c = jax.lax.fori_loop(0, _N_ITERS - 1, body, c0)

    _, prods = half_steps(c)
    for mm in range(_BLOCK_B):
        v = jnp.sum(prods[mm], axis=0, keepdims=True)
        cf = 1.0 / jnp.maximum(v, _TINY)
        o_ref[mm] = prods[mm] * cf


def _sinkhorn_pallas(x):
    b, n, _ = x.shape
    grid = (b // _BLOCK_B,)
    return pl.pallas_call(
        _sinkhorn_block,
        out_shape=jax.ShapeDtypeStruct(x.shape, x.dtype),
        grid=grid,
        in_specs=[pl.BlockSpec((_BLOCK_B, n, n), lambda i: (i, 0, 0))],
        out_specs=pl.BlockSpec((_BLOCK_B, n, n), lambda i: (i, 0, 0)),
        compiler_params=pltpu.CompilerParams(
            dimension_semantics=("parallel",),
        ),
        name="sinkhorn",
    )(x)


def kernel(input_tensor):
    devs = jax.devices()
    b = input_tensor.shape[0]
    nd = len(devs)
    while nd > 1 and b % (nd * _BLOCK_B) != 0:
        nd -= 1
    if nd <= 1:
        return _sinkhorn_pallas(input_tensor)
    mesh = jax.sharding.Mesh(devs[:nd], ("b",))
    pspec = jax.sharding.PartitionSpec("b")
    fn = jax.shard_map(
        _sinkhorn_pallas, mesh=mesh, in_specs=pspec, out_specs=pspec,
        check_vma=False,
    )
    return fn(input_tensor)


# speedup vs baseline: 1.1763x; 1.1763x over previous
import jax
import jax.numpy as jnp
from jax.experimental import pallas as pl
from jax.experimental.pallas import tpu as pltpu

_N_ITERS = 21
_INV_TEMPERATURE = 25.0
_TINY = 1e-30
_BLOCK_B = 16


def _sinkhorn_block(x_ref, o_ref):
    for mm in range(_BLOCK_B):
        la = x_ref[mm] * _INV_TEMPERATURE
        m = jnp.max(la, axis=1, keepdims=True)
        o_ref[mm] = jnp.exp(la - m)

    def half_steps(c):
        rs, prods = [], []
        for mm in range(_BLOCK_B):
            k = o_ref[mm]
            u = jnp.sum(k * c[mm], axis=1, keepdims=True)
            r = 1.0 / jnp.maximum(u, _TINY)
            rs.append(r)
            prods.append(k * r)
        return rs, prods

    def body(_, c):
        _, prods = half_steps(c)
        news = []
        for mm in range(_BLOCK_B):
            v = jnp.sum(prods[mm], axis=0)
            news.append(1.0 / jnp.maximum(v, _TINY))
        return jnp.stack(news)

    c0 = jnp.ones((_BLOCK_B, 128), jnp.float32)
    c = jax.lax.fori_loop(0, _N_ITERS - 1, body, c0)

    _, prods = half_steps(c)
    for mm in range(_BLOCK_B):
        v = jnp.sum(prods[mm], axis=0, keepdims=True)
        cf = 1.0 / jnp.maximum(v, _TINY)
        o_ref[mm] = prods[mm] * cf


def _sinkhorn_pallas(x):
    b, n, _ = x.shape
    grid = (b // _BLOCK_B,)
    return pl.pallas_call(
        _sinkhorn_block,
        out_shape=jax.ShapeDtypeStruct(x.shape, x.dtype),
        grid=grid,
        in_specs=[pl.BlockSpec((_BLOCK_B, n, n), lambda i: (i, 0, 0))],
        out_specs=pl.BlockSpec((_BLOCK_B, n, n), lambda i: (i, 0, 0)),
        compiler_params=pltpu.CompilerParams(
            dimension_semantics=("parallel",),
        ),
        name="sinkhorn",
    )(x)


def kernel(input_tensor):
    devs = jax.devices()
    b = input_tensor.shape[0]
    nd = len(devs)
    while nd > 1 and b % (nd * _BLOCK_B) != 0:
        nd -= 1
    if nd <= 1:
        return _sinkhorn_pallas(input_tensor)
    mesh = jax.sharding.Mesh(devs[:nd], ("b",))
    pspec = jax.sharding.PartitionSpec("b")
    fn = jax.shard_map(
        _sinkhorn_pallas, mesh=mesh, in_specs=pspec, out_specs=pspec,
        check_vma=False,
    )
    return fn(input_tensor)


# speedup vs baseline: 1.3017x; 1.1066x over previous
import jax
import jax.numpy as jnp
from jax.experimental import pallas as pl
from jax.experimental.pallas import tpu as pltpu

_N_ITERS = 21
_INV_TEMPERATURE = 25.0
_TINY = 1e-30
_BLOCK_B = 32


def _sinkhorn_block(x_ref, o_ref):
    for mm in range(_BLOCK_B):
        la = x_ref[mm] * _INV_TEMPERATURE
        m = jnp.max(la, axis=1, keepdims=True)
        o_ref[mm] = jnp.exp(la - m)

    def half_steps(c):
        rs, prods = [], []
        for mm in range(_BLOCK_B):
            k = o_ref[mm]
            u = jnp.sum(k * c[mm], axis=1, keepdims=True)
            r = 1.0 / jnp.maximum(u, _TINY)
            rs.append(r)
            prods.append(k * r)
        return rs, prods

    def body(_, c):
        _, prods = half_steps(c)
        news = []
        for mm in range(_BLOCK_B):
            v = jnp.sum(prods[mm], axis=0)
            news.append(1.0 / jnp.maximum(v, _TINY))
        return jnp.stack(news)

    c0 = jnp.ones((_BLOCK_B, 128), jnp.float32)
    c = jax.lax.fori_loop(0, _N_ITERS - 1, body, c0)

    _, prods = half_steps(c)
    for mm in range(_BLOCK_B):
        v = jnp.sum(prods[mm], axis=0, keepdims=True)
        cf = 1.0 / jnp.maximum(v, _TINY)
        o_ref[mm] = prods[mm] * cf


def _sinkhorn_pallas(x):
    b, n, _ = x.shape
    grid = (b // _BLOCK_B,)
    return pl.pallas_call(
        _sinkhorn_block,
        out_shape=jax.ShapeDtypeStruct(x.shape, x.dtype),
        grid=grid,
        in_specs=[pl.BlockSpec((_BLOCK_B, n, n), lambda i: (i, 0, 0))],
        out_specs=pl.BlockSpec((_BLOCK_B, n, n), lambda i: (i, 0, 0)),
        compiler_params=pltpu.CompilerParams(
            dimension_semantics=("parallel",),
        ),
        name="sinkhorn",
    )(x)


def kernel(input_tensor):
    devs = jax.devices()
    b = input_tensor.shape[0]
    nd = len(devs)
    while nd > 1 and b % (nd * _BLOCK_B) != 0:
        nd -= 1
    if nd <= 1:
        return _sinkhorn_pallas(input_tensor)
    mesh = jax.sharding.Mesh(devs[:nd], ("b",))
    pspec = jax.sharding.PartitionSpec("b")
    fn = jax.shard_map(
        _sinkhorn_pallas, mesh=mesh, in_specs=pspec, out_specs=pspec,
        check_vma=False,
    )
    return fn(input_tensor)


# speedup vs baseline: 1.3741x; 1.0556x over previous
import jax
import jax.numpy as jnp
from jax.experimental import pallas as pl
from jax.experimental.pallas import tpu as pltpu

_N_ITERS = 21
_INV_TEMPERATURE = 25.0
_TINY = 1e-30
_BLOCK_B = 64


def _sinkhorn_block(x_ref, o_ref):
    for mm in range(_BLOCK_B):
        la = x_ref[mm] * _INV_TEMPERATURE
        m = jnp.max(la, axis=1, keepdims=True)
        o_ref[mm] = jnp.exp(la - m)

    def half_steps(c):
        rs, prods = [], []
        for mm in range(_BLOCK_B):
            k = o_ref[mm]
            u = jnp.sum(k * c[mm], axis=1, keepdims=True)
            r = 1.0 / jnp.maximum(u, _TINY)
            rs.append(r)
            prods.append(k * r)
        return rs, prods

    def body(_, c):
        _, prods = half_steps(c)
        news = []
        for mm in range(_BLOCK_B):
            v = jnp.sum(prods[mm], axis=0)
            news.append(1.0 / jnp.maximum(v, _TINY))
        return jnp.stack(news)

    c0 = jnp.ones((_BLOCK_B, 128), jnp.float32)
    c = jax.lax.fori_loop(0, _N_ITERS - 1, body, c0)

    _, prods = half_steps(c)
    for mm in range(_BLOCK_B):
        v = jnp.sum(prods[mm], axis=0, keepdims=True)
        cf = 1.0 / jnp.maximum(v, _TINY)
        o_ref[mm] = prods[mm] * cf


def _sinkhorn_pallas(x):
    b, n, _ = x.shape
    grid = (b // _BLOCK_B,)
    return pl.pallas_call(
        _sinkhorn_block,
        out_shape=jax.ShapeDtypeStruct(x.shape, x.dtype),
        grid=grid,
        in_specs=[pl.BlockSpec((_BLOCK_B, n, n), lambda i: (i, 0, 0))],
        out_specs=pl.BlockSpec((_BLOCK_B, n, n), lambda i: (i, 0, 0)),
        compiler_params=pltpu.CompilerParams(
            dimension_semantics=("parallel",),
        ),
        name="sinkhorn",
    )(x)


def kernel(input_tensor):
    devs = jax.devices()
    b = input_tensor.shape[0]
    nd = len(devs)
    while nd > 1 and b % (nd * _BLOCK_B) != 0:
        nd -= 1
    if nd <= 1:
        return _sinkhorn_pallas(input_tensor)
    mesh = jax.sharding.Mesh(devs[:nd], ("b",))
    pspec = jax.sharding.PartitionSpec("b")
    fn = jax.shard_map(
        _sinkhorn_pallas, mesh=mesh, in_specs=pspec, out_specs=pspec,
        check_vma=False,
    )
    return fn(input_tensor)


# speedup vs baseline: 1.4095x; 1.0258x over previous
import jax
import jax.numpy as jnp
from jax.experimental import pallas as pl
from jax.experimental.pallas import tpu as pltpu

_N_ITERS = 21
_N_BF16 = 17
_INV_TEMPERATURE = 25.0
_TINY = 1e-30
_BLOCK_B = 64


def _iter_once(kref, c, dt):
    tiny = jnp.asarray(_TINY, dt)
    one = jnp.asarray(1.0, dt)
    news = []
    for mm in range(_BLOCK_B):
        k = kref[mm]
        u = jnp.sum(k * c[mm], axis=1, keepdims=True, dtype=dt)
        r = one / jnp.maximum(u, tiny)
        v = jnp.sum(k * r, axis=0, dtype=dt)
        news.append(one / jnp.maximum(v, tiny))
    return jnp.stack(news)


def _sinkhorn_block(x_ref, o_ref, kb_ref):
    for mm in range(_BLOCK_B):
        la = x_ref[mm] * _INV_TEMPERATURE
        m = jnp.max(la, axis=1, keepdims=True)
        k = jnp.exp(la - m)
        o_ref[mm] = k
        kb_ref[mm] = k.astype(jnp.bfloat16)

    cb = jnp.ones((_BLOCK_B, 128), jnp.bfloat16)
    cb = jax.lax.fori_loop(
        0, _N_BF16, lambda i, c: _iter_once(kb_ref, c, jnp.bfloat16), cb)
    c = cb.astype(jnp.float32)
    c = jax.lax.fori_loop(
        0, _N_ITERS - _N_BF16 - 1,
        lambda i, c: _iter_once(o_ref, c, jnp.float32), c)

    for mm in range(_BLOCK_B):
        k = o_ref[mm]
        u = jnp.sum(k * c[mm], axis=1, keepdims=True)
        r = 1.0 / jnp.maximum(u, _TINY)
        s = k * r
        v = jnp.sum(s, axis=0, keepdims=True)
        cf = 1.0 / jnp.maximum(v, _TINY)
        o_ref[mm] = s * cf


def _sinkhorn_pallas(x):
    b, n, _ = x.shape
    grid = (b // _BLOCK_B,)
    return pl.pallas_call(
        _sinkhorn_block,
        out_shape=jax.ShapeDtypeStruct(x.shape, x.dtype),
        grid=grid,
        in_specs=[pl.BlockSpec((_BLOCK_B, n, n), lambda i: (i, 0, 0))],
        out_specs=pl.BlockSpec((_BLOCK_B, n, n), lambda i: (i, 0, 0)),
        scratch_shapes=[pltpu.VMEM((_BLOCK_B, n, n), jnp.bfloat16)],
        compiler_params=pltpu.CompilerParams(
            dimension_semantics=("parallel",),
        ),
        name="sinkhorn",
    )(x)


def kernel(input_tensor):
    devs = jax.devices()
    b = input_tensor.shape[0]
    nd = len(devs)
    while nd > 1 and b % (nd * _BLOCK_B) != 0:
        nd -= 1
    if nd <= 1:
        return _sinkhorn_pallas(input_tensor)
    mesh = jax.sharding.Mesh(devs[:nd], ("b",))
    pspec = jax.sharding.PartitionSpec("b")
    fn = jax.shard_map(
        _sinkhorn_pallas, mesh=mesh, in_specs=pspec, out_specs=pspec,
        check_vma=False,
    )
    return fn(input_tensor)
